# baseline (speedup 1.0000x reference)
import functools
import math

import jax
import jax.numpy as jnp
import numpy as np
from jax import lax
from jax.experimental import pallas as pl
from jax.experimental.pallas import tpu as pltpu
from jax.experimental.pallas import tpu_sc as plsc

_CAPACITY = 512
_NEG_INF = -1e30


def _pos_encoding(seqlen, emsize):
    pos = np.arange(seqlen, dtype=np.float64)[:, None]
    j = np.arange(emsize, dtype=np.float64)[None, :]
    p = pos / np.power(10000.0, 2.0 * np.floor(j / 2.0) / emsize)
    p[:, 0::2] = np.sin(p[:, 0::2])
    p[:, 1::2] = np.cos(p[:, 1::2])
    return jnp.asarray(p, dtype=jnp.float32)


def _layer_norm(x, g, b, eps=1e-5):
    mu = jnp.mean(x, axis=-1, keepdims=True)
    var = jnp.mean((x - mu) ** 2, axis=-1, keepdims=True)
    return (x - mu) / jnp.sqrt(var + eps) * g + b


def _mha(x, in_w, in_b, out_w, out_b, nheads, mask):
    B, S, d = x.shape
    dh = d // nheads
    qkv = x @ in_w.T + in_b
    q, k, v = jnp.split(qkv, 3, axis=-1)

    def sh(t):
        return t.reshape(B, S, nheads, dh).transpose(0, 2, 1, 3)

    q, k, v = sh(q), sh(k), sh(v)
    scores = (q @ k.transpose(0, 1, 3, 2)) / math.sqrt(dh) + mask
    a = jax.nn.softmax(scores, axis=-1)
    o = (a @ v).transpose(0, 2, 1, 3).reshape(B, S, d)
    return o @ out_w.T + out_b


def _sc_gather(table, idx):
    V, D = table.shape
    B = idx.shape[0]
    info = plsc.get_sparse_core_info()
    nw = info.num_cores * info.num_subcores
    b_per_w = B // nw
    mesh = plsc.VectorSubcoreMesh(core_axis_name="c", subcore_axis_name="s")

    @functools.partial(
        pl.kernel,
        mesh=mesh,
        out_type=jax.ShapeDtypeStruct((B, D), jnp.float32),
        scratch_types=[
            pltpu.VMEM((b_per_w,), jnp.int32),
            pltpu.VMEM((b_per_w, D), jnp.float32),
            pltpu.SemaphoreType.DMA,
        ],
    )
    def k(table_hbm, idx_hbm, out_hbm, idx_v, rows_v, sem):
        wid = lax.axis_index("s") * info.num_cores + lax.axis_index("c")
        base = wid * b_per_w
        pltpu.sync_copy(idx_hbm.at[pl.ds(base, b_per_w)], idx_v)
        pltpu.async_copy(table_hbm.at[idx_v], rows_v, sem).wait()
        pltpu.sync_copy(rows_v, out_hbm.at[pl.ds(base, b_per_w)])

    return k(table, idx)


def _routing_body(x_ref, gw_ref, ep_ref, gs_ref):
    S = x_ref.shape[0]
    E = gw_ref.shape[1]
    logits = jnp.dot(x_ref[...], gw_ref[...], preferred_element_type=jnp.float32)
    m = jnp.max(logits, axis=1, keepdims=True)
    ex = jnp.exp(logits - m)
    gates = ex / jnp.sum(ex, axis=1, keepdims=True)

    eidx = jax.lax.broadcasted_iota(jnp.int32, (S, E), 1)
    g0 = jnp.max(gates, axis=1, keepdims=True)
    e0 = jnp.min(jnp.where(gates == g0, eidx, E), axis=1, keepdims=True)
    masked = jnp.where(eidx == e0, _NEG_INF, gates)
    g1 = jnp.max(masked, axis=1, keepdims=True)
    e1 = jnp.min(jnp.where(masked == g1, eidx, E), axis=1, keepdims=True)

    m0 = (eidx == e0).astype(jnp.float32)
    m1 = (eidx == e1).astype(jnp.float32)

    r = jax.lax.broadcasted_iota(jnp.int32, (S, S), 0)
    c = jax.lax.broadcasted_iota(jnp.int32, (S, S), 1)
    L = (r >= c).astype(jnp.float32)
    c0 = jnp.dot(L, m0, preferred_element_type=jnp.float32)
    c1 = jnp.dot(L, m1, preferred_element_type=jnp.float32)
    p0 = jnp.sum(c0 * m0, axis=1, keepdims=True)
    e1cnt = jnp.sum(c0 * m0, axis=0, keepdims=True)
    p1 = jnp.sum(c1 * m1, axis=1, keepdims=True) + jnp.sum(
        e1cnt * m1, axis=1, keepdims=True
    )

    keep0 = (p0 < _CAPACITY).astype(jnp.float32)
    keep1 = (p1 < _CAPACITY).astype(jnp.float32)
    ep_ref[:, 0:1] = e0.astype(jnp.float32)
    ep_ref[:, 1:2] = e1.astype(jnp.float32)
    ep_ref[:, 2:3] = p0 * keep0
    ep_ref[:, 3:4] = p1 * keep1
    gs_ref[:, 0:1] = g0 * keep0
    gs_ref[:, 1:2] = g1 * keep1


def _routing_call(xh, gate_w):
    S = xh.shape[0]
    return pl.pallas_call(
        _routing_body,
        out_shape=(
            jax.ShapeDtypeStruct((S, 4), jnp.float32),
            jax.ShapeDtypeStruct((S, 2), jnp.float32),
        ),
    )(xh, gate_w)


def _moe_body(x_ref, ep_ref, gs_ref, w1_ref, w2_ref, o_ref):
    e = pl.program_id(0)
    S = x_ref.shape[0]
    ef = jnp.float32(e)
    e0 = ep_ref[:, 0:1]
    e1 = ep_ref[:, 1:2]
    p0 = ep_ref[:, 2:3]
    p1 = ep_ref[:, 3:4]
    g0 = gs_ref[:, 0:1]
    g1 = gs_ref[:, 1:2]

    cap = jax.lax.broadcasted_iota(jnp.int32, (S, _CAPACITY), 1).astype(jnp.float32)
    s0 = jnp.where((e0 == ef) & (p0 == cap) & (p0 > 0.0), 1.0, 0.0)
    s1 = jnp.where((e1 == ef) & (p1 == cap) & (p1 > 0.0), 1.0, 0.0)
    disp = s0 + s1
    comb = g0 * s0 + g1 * s1

    ein = jax.lax.dot_general(
        disp, x_ref[...], (((0,), (0,)), ((), ())),
        preferred_element_type=jnp.float32,
    )
    h = jnp.maximum(
        jnp.dot(ein, w1_ref[0], preferred_element_type=jnp.float32), 0.0
    )
    eo = jnp.dot(h, w2_ref[0], preferred_element_type=jnp.float32)
    part = jnp.dot(comb, eo, preferred_element_type=jnp.float32)

    @pl.when(e == 0)
    def _():
        o_ref[...] = jnp.zeros_like(o_ref)

    o_ref[...] += part


def _moe_call(xh, ep, gs, w1, w2):
    S, d = xh.shape
    E, _, H = w1.shape
    return pl.pallas_call(
        _moe_body,
        grid=(E,),
        in_specs=[
            pl.BlockSpec((S, d), lambda e: (0, 0)),
            pl.BlockSpec((S, 4), lambda e: (0, 0)),
            pl.BlockSpec((S, 2), lambda e: (0, 0)),
            pl.BlockSpec((1, d, H), lambda e: (e, 0, 0)),
            pl.BlockSpec((1, H, d), lambda e: (e, 0, 0)),
        ],
        out_specs=pl.BlockSpec((S, d), lambda e: (0, 0)),
        out_shape=jax.ShapeDtypeStruct((S, d), jnp.float32),
    )(xh, ep, gs, w1, w2)


def _dec_body(h_ref, w_ref, b_ref, mo_ref, so_ref,
              ma_ref, sa_ref, mb_ref, sb_ref):
    j = pl.program_id(1)
    nj = pl.num_programs(1)
    S = h_ref.shape[0]
    VT = w_ref.shape[0]
    HT = VT // 2

    @pl.when(j == 0)
    def _():
        ma_ref[...] = jnp.full((1, S), _NEG_INF, jnp.float32)
        sa_ref[...] = jnp.zeros((1, S), jnp.float32)
        mb_ref[...] = jnp.full((1, S), _NEG_INF, jnp.float32)
        sb_ref[...] = jnp.zeros((1, S), jnp.float32)

    hb = h_ref[...].astype(jnp.bfloat16)
    bias = jnp.reshape(b_ref[...], (VT, 1))

    def half(lo, m_ref_, s_ref_):
        lg = jax.lax.dot_general(
            w_ref[lo:lo + HT, :].astype(jnp.bfloat16), hb,
            (((1,), (1,)), ((), ())),
            preferred_element_type=jnp.float32,
        )
        lg = lg + bias[lo:lo + HT, :]
        tile_max = jnp.max(lg, axis=0, keepdims=True)
        m_old = m_ref_[...]
        m_new = jnp.maximum(m_old, tile_max)
        rs = jnp.sum(jnp.exp(lg - m_new), axis=0, keepdims=True)
        s_ref_[...] = s_ref_[...] * jnp.exp(m_old - m_new) + rs
        m_ref_[...] = m_new

    half(0, ma_ref, sa_ref)
    half(HT, mb_ref, sb_ref)

    @pl.when(j == nj - 1)
    def _():
        ma, sa = ma_ref[...], sa_ref[...]
        mb, sb = mb_ref[...], sb_ref[...]
        m = jnp.maximum(ma, mb)
        s = sa * jnp.exp(ma - m) + sb * jnp.exp(mb - m)
        mo_ref[0] = m
        so_ref[0] = s


def _decoder_call(h2, dec_w, dec_b, y):
    S, d = h2.shape
    V = dec_w.shape[0]
    VT = 2000 if V % 2000 == 0 else V
    nsplit = 2 if (V // VT) % 2 == 0 else 1
    grid = V // VT // nsplit
    mo, so = pl.pallas_call(
        _dec_body,
        grid=(nsplit, grid),
        in_specs=[
            pl.BlockSpec((S, d), lambda g, j: (0, 0)),
            pl.BlockSpec((VT, d), lambda g, j: (g * (V // VT // 2) + j, 0)
                         if V // VT > 1 else (0, 0)),
            pl.BlockSpec((1, 1, VT), lambda g, j: (g * (V // VT // 2) + j, 0, 0)
                         if V // VT > 1 else (0, 0, 0)),
        ],
        out_specs=(
            pl.BlockSpec((1, 1, S), lambda g, j: (g, 0, 0)),
            pl.BlockSpec((1, 1, S), lambda g, j: (g, 0, 0)),
        ),
        out_shape=(
            jax.ShapeDtypeStruct((nsplit, 1, S), jnp.float32),
            jax.ShapeDtypeStruct((nsplit, 1, S), jnp.float32),
        ),
        scratch_shapes=[
            pltpu.VMEM((1, S), jnp.float32),
            pltpu.VMEM((1, S), jnp.float32),
            pltpu.VMEM((1, S), jnp.float32),
            pltpu.VMEM((1, S), jnp.float32),
        ],
        compiler_params=pltpu.CompilerParams(
            dimension_semantics=("parallel", "arbitrary"),
        ),
    )(h2, dec_w, jnp.reshape(dec_b, (V // VT, 1, VT)))
    m = jnp.max(mo, axis=0)
    s = jnp.sum(so * jnp.exp(mo - m[None]), axis=0)
    sum_lse = jnp.sum(m + jnp.log(s))
    rows = _sc_gather(dec_w, y)
    tgt = jnp.sum(h2 * rows, axis=1) + jnp.take(dec_b, y)
    return sum_lse - jnp.sum(tgt)


def kernel(x, y, emb, l0_in_w, l0_in_b, l0_out_w, l0_out_b, l0_ff1_w, l0_ff1_b,
           l0_ff2_w, l0_ff2_b, l0_n1_g, l0_n1_b, l0_n2_g, l0_n2_b, l1_in_w,
           l1_in_b, l1_out_w, l1_out_b, l1_gate, l1_w1, l1_w2, l1_n1_g,
           l1_n1_b, l1_n2_g, l1_n2_b, dec_w, dec_b):
    B, S = x.shape
    d = emb.shape[1]
    nheads = 12

    h = _sc_gather(emb, x[0])[None] * math.sqrt(d)
    h = h + _pos_encoding(S, d)[None]
    mask = jnp.triu(jnp.full((S, S), -jnp.inf, dtype=jnp.float32), k=1)

    h = _layer_norm(h + _mha(h, l0_in_w, l0_in_b, l0_out_w, l0_out_b, nheads, mask),
                    l0_n1_g, l0_n1_b)
    ff = jax.nn.relu(h @ l0_ff1_w.T + l0_ff1_b) @ l0_ff2_w.T + l0_ff2_b
    h = _layer_norm(h + ff, l0_n2_g, l0_n2_b)

    h = _layer_norm(h + _mha(h, l1_in_w, l1_in_b, l1_out_w, l1_out_b, nheads, mask),
                    l1_n1_g, l1_n1_b)
    xh = h[0]
    ep, gs = _routing_call(xh, l1_gate)
    moe = _moe_call(xh, ep, gs, l1_w1, l1_w2)
    h = _layer_norm(h + moe[None], l1_n2_g, l1_n2_b)

    nll = _decoder_call(h[0], dec_w, dec_b, y[0])
    return nll


# speedup vs baseline: 1.0240x; 1.0240x over previous
import functools
import math

import jax
import jax.numpy as jnp
import numpy as np
from jax import lax
from jax.experimental import pallas as pl
from jax.experimental.pallas import tpu as pltpu
from jax.experimental.pallas import tpu_sc as plsc

_CAPACITY = 512
_NEG_INF = -1e30


def _pos_encoding(seqlen, emsize):
    pos = np.arange(seqlen, dtype=np.float64)[:, None]
    j = np.arange(emsize, dtype=np.float64)[None, :]
    p = pos / np.power(10000.0, 2.0 * np.floor(j / 2.0) / emsize)
    p[:, 0::2] = np.sin(p[:, 0::2])
    p[:, 1::2] = np.cos(p[:, 1::2])
    return jnp.asarray(p, dtype=jnp.float32)


def _layer_norm(x, g, b, eps=1e-5):
    mu = jnp.mean(x, axis=-1, keepdims=True)
    var = jnp.mean((x - mu) ** 2, axis=-1, keepdims=True)
    return (x - mu) / jnp.sqrt(var + eps) * g + b


def _bmm(a, b):
    nb = a.ndim - 1
    batch = tuple(range(a.ndim - 2))
    return jax.lax.dot_general(
        a.astype(jnp.bfloat16), b.astype(jnp.bfloat16),
        (((nb,), (a.ndim - 2,) if b.ndim == a.ndim else (0,)), (batch, batch)
         if b.ndim == a.ndim else ((), ())),
        preferred_element_type=jnp.float32,
    )


def _mha(x, in_w, in_b, out_w, out_b, nheads, mask):
    B, S, d = x.shape
    dh = d // nheads
    qkv = _bmm(x, in_w.T) + in_b
    q, k, v = jnp.split(qkv, 3, axis=-1)

    def sh(t):
        return t.reshape(B, S, nheads, dh).transpose(0, 2, 1, 3)

    q, k, v = sh(q), sh(k), sh(v)
    scores = _bmm(q, k.transpose(0, 1, 3, 2)) / math.sqrt(dh) + mask
    a = jax.nn.softmax(scores, axis=-1)
    o = _bmm(a, v).transpose(0, 2, 1, 3).reshape(B, S, d)
    return _bmm(o, out_w.T) + out_b


def _sc_gather(table, idx):
    V, D = table.shape
    B = idx.shape[0]
    info = plsc.get_sparse_core_info()
    nw = info.num_cores * info.num_subcores
    b_per_w = B // nw
    mesh = plsc.VectorSubcoreMesh(core_axis_name="c", subcore_axis_name="s")

    @functools.partial(
        pl.kernel,
        mesh=mesh,
        out_type=jax.ShapeDtypeStruct((B, D), jnp.float32),
        scratch_types=[
            pltpu.VMEM((b_per_w,), jnp.int32),
            pltpu.VMEM((b_per_w, D), jnp.float32),
            pltpu.SemaphoreType.DMA,
        ],
    )
    def k(table_hbm, idx_hbm, out_hbm, idx_v, rows_v, sem):
        wid = lax.axis_index("s") * info.num_cores + lax.axis_index("c")
        base = wid * b_per_w
        pltpu.sync_copy(idx_hbm.at[pl.ds(base, b_per_w)], idx_v)
        pltpu.async_copy(table_hbm.at[idx_v], rows_v, sem).wait()
        pltpu.sync_copy(rows_v, out_hbm.at[pl.ds(base, b_per_w)])

    return k(table, idx)


def _routing_body(x_ref, gw_ref, ep_ref, gs_ref):
    S = x_ref.shape[0]
    E = gw_ref.shape[1]
    logits = jnp.dot(x_ref[...], gw_ref[...], preferred_element_type=jnp.float32)
    m = jnp.max(logits, axis=1, keepdims=True)
    ex = jnp.exp(logits - m)
    gates = ex / jnp.sum(ex, axis=1, keepdims=True)

    eidx = jax.lax.broadcasted_iota(jnp.int32, (S, E), 1)
    g0 = jnp.max(gates, axis=1, keepdims=True)
    e0 = jnp.min(jnp.where(gates == g0, eidx, E), axis=1, keepdims=True)
    masked = jnp.where(eidx == e0, _NEG_INF, gates)
    g1 = jnp.max(masked, axis=1, keepdims=True)
    e1 = jnp.min(jnp.where(masked == g1, eidx, E), axis=1, keepdims=True)

    m0 = (eidx == e0).astype(jnp.float32)
    m1 = (eidx == e1).astype(jnp.float32)

    r = jax.lax.broadcasted_iota(jnp.int32, (S, S), 0)
    c = jax.lax.broadcasted_iota(jnp.int32, (S, S), 1)
    L = (r >= c).astype(jnp.float32)
    c0 = jnp.dot(L, m0, preferred_element_type=jnp.float32)
    c1 = jnp.dot(L, m1, preferred_element_type=jnp.float32)
    p0 = jnp.sum(c0 * m0, axis=1, keepdims=True)
    e1cnt = jnp.sum(c0 * m0, axis=0, keepdims=True)
    p1 = jnp.sum(c1 * m1, axis=1, keepdims=True) + jnp.sum(
        e1cnt * m1, axis=1, keepdims=True
    )

    keep0 = (p0 < _CAPACITY).astype(jnp.float32)
    keep1 = (p1 < _CAPACITY).astype(jnp.float32)
    ep_ref[:, 0:1] = e0.astype(jnp.float32)
    ep_ref[:, 1:2] = e1.astype(jnp.float32)
    ep_ref[:, 2:3] = p0 * keep0
    ep_ref[:, 3:4] = p1 * keep1
    gs_ref[:, 0:1] = g0 * keep0
    gs_ref[:, 1:2] = g1 * keep1


def _routing_call(xh, gate_w):
    S = xh.shape[0]
    return pl.pallas_call(
        _routing_body,
        out_shape=(
            jax.ShapeDtypeStruct((S, 4), jnp.float32),
            jax.ShapeDtypeStruct((S, 2), jnp.float32),
        ),
    )(xh, gate_w)


def _moe_body(x_ref, ep_ref, gs_ref, w1_ref, w2_ref, o_ref):
    e = pl.program_id(0)
    S = x_ref.shape[0]
    ef = jnp.float32(e)
    e0 = ep_ref[:, 0:1]
    e1 = ep_ref[:, 1:2]
    p0 = ep_ref[:, 2:3]
    p1 = ep_ref[:, 3:4]
    g0 = gs_ref[:, 0:1]
    g1 = gs_ref[:, 1:2]

    cap = jax.lax.broadcasted_iota(jnp.int32, (S, _CAPACITY), 1).astype(jnp.float32)
    s0 = jnp.where((e0 == ef) & (p0 == cap) & (p0 > 0.0), 1.0, 0.0)
    s1 = jnp.where((e1 == ef) & (p1 == cap) & (p1 > 0.0), 1.0, 0.0)
    disp = s0 + s1
    comb = g0 * s0 + g1 * s1

    ein = jax.lax.dot_general(
        disp, x_ref[...], (((0,), (0,)), ((), ())),
        preferred_element_type=jnp.float32,
    )
    h = jnp.maximum(
        jax.lax.dot_general(
            ein.astype(jnp.bfloat16), w1_ref[0].astype(jnp.bfloat16),
            (((1,), (0,)), ((), ())), preferred_element_type=jnp.float32),
        0.0,
    )
    eo = jax.lax.dot_general(
        h.astype(jnp.bfloat16), w2_ref[0].astype(jnp.bfloat16),
        (((1,), (0,)), ((), ())), preferred_element_type=jnp.float32,
    )
    part = jnp.dot(comb, eo, preferred_element_type=jnp.float32)

    @pl.when(e == 0)
    def _():
        o_ref[...] = jnp.zeros_like(o_ref)

    o_ref[...] += part


def _moe_call(xh, ep, gs, w1, w2):
    S, d = xh.shape
    E, _, H = w1.shape
    return pl.pallas_call(
        _moe_body,
        grid=(E,),
        in_specs=[
            pl.BlockSpec((S, d), lambda e: (0, 0)),
            pl.BlockSpec((S, 4), lambda e: (0, 0)),
            pl.BlockSpec((S, 2), lambda e: (0, 0)),
            pl.BlockSpec((1, d, H), lambda e: (e, 0, 0)),
            pl.BlockSpec((1, H, d), lambda e: (e, 0, 0)),
        ],
        out_specs=pl.BlockSpec((S, d), lambda e: (0, 0)),
        out_shape=jax.ShapeDtypeStruct((S, d), jnp.float32),
    )(xh, ep, gs, w1, w2)


def _dec_body(h_ref, w_ref, b_ref, mo_ref, so_ref,
              ma_ref, sa_ref, mb_ref, sb_ref):
    j = pl.program_id(1)
    nj = pl.num_programs(1)
    S = h_ref.shape[0]
    VT = w_ref.shape[0]
    HT = VT // 2

    @pl.when(j == 0)
    def _():
        ma_ref[...] = jnp.full((1, S), _NEG_INF, jnp.float32)
        sa_ref[...] = jnp.zeros((1, S), jnp.float32)
        mb_ref[...] = jnp.full((1, S), _NEG_INF, jnp.float32)
        sb_ref[...] = jnp.zeros((1, S), jnp.float32)

    hb = h_ref[...].astype(jnp.bfloat16)
    bias = jnp.reshape(b_ref[...], (VT, 1))

    def half(lo, m_ref_, s_ref_):
        lg = jax.lax.dot_general(
            w_ref[lo:lo + HT, :].astype(jnp.bfloat16), hb,
            (((1,), (1,)), ((), ())),
            preferred_element_type=jnp.float32,
        )
        lg = lg + bias[lo:lo + HT, :]
        tile_max = jnp.max(lg, axis=0, keepdims=True)
        m_old = m_ref_[...]
        m_new = jnp.maximum(m_old, tile_max)
        rs = jnp.sum(jnp.exp(lg - m_new), axis=0, keepdims=True)
        s_ref_[...] = s_ref_[...] * jnp.exp(m_old - m_new) + rs
        m_ref_[...] = m_new

    half(0, ma_ref, sa_ref)
    half(HT, mb_ref, sb_ref)

    @pl.when(j == nj - 1)
    def _():
        ma, sa = ma_ref[...], sa_ref[...]
        mb, sb = mb_ref[...], sb_ref[...]
        m = jnp.maximum(ma, mb)
        s = sa * jnp.exp(ma - m) + sb * jnp.exp(mb - m)
        mo_ref[0] = m
        so_ref[0] = s


def _decoder_call(h2, dec_w, dec_b, y):
    S, d = h2.shape
    V = dec_w.shape[0]
    VT = 2000 if V % 2000 == 0 else V
    nsplit = 2 if (V // VT) % 2 == 0 else 1
    grid = V // VT // nsplit
    mo, so = pl.pallas_call(
        _dec_body,
        grid=(nsplit, grid),
        in_specs=[
            pl.BlockSpec((S, d), lambda g, j: (0, 0)),
            pl.BlockSpec((VT, d), lambda g, j: (g * (V // VT // 2) + j, 0)
                         if V // VT > 1 else (0, 0)),
            pl.BlockSpec((1, 1, VT), lambda g, j: (g * (V // VT // 2) + j, 0, 0)
                         if V // VT > 1 else (0, 0, 0)),
        ],
        out_specs=(
            pl.BlockSpec((1, 1, S), lambda g, j: (g, 0, 0)),
            pl.BlockSpec((1, 1, S), lambda g, j: (g, 0, 0)),
        ),
        out_shape=(
            jax.ShapeDtypeStruct((nsplit, 1, S), jnp.float32),
            jax.ShapeDtypeStruct((nsplit, 1, S), jnp.float32),
        ),
        scratch_shapes=[
            pltpu.VMEM((1, S), jnp.float32),
            pltpu.VMEM((1, S), jnp.float32),
            pltpu.VMEM((1, S), jnp.float32),
            pltpu.VMEM((1, S), jnp.float32),
        ],
        compiler_params=pltpu.CompilerParams(
            dimension_semantics=("parallel", "arbitrary"),
        ),
    )(h2, dec_w, jnp.reshape(dec_b, (V // VT, 1, VT)))
    m = jnp.max(mo, axis=0)
    s = jnp.sum(so * jnp.exp(mo - m[None]), axis=0)
    sum_lse = jnp.sum(m + jnp.log(s))
    rows = _sc_gather(dec_w, y)
    tgt = jnp.sum(h2 * rows, axis=1) + jnp.take(dec_b, y)
    return sum_lse - jnp.sum(tgt)


def kernel(x, y, emb, l0_in_w, l0_in_b, l0_out_w, l0_out_b, l0_ff1_w, l0_ff1_b,
           l0_ff2_w, l0_ff2_b, l0_n1_g, l0_n1_b, l0_n2_g, l0_n2_b, l1_in_w,
           l1_in_b, l1_out_w, l1_out_b, l1_gate, l1_w1, l1_w2, l1_n1_g,
           l1_n1_b, l1_n2_g, l1_n2_b, dec_w, dec_b):
    B, S = x.shape
    d = emb.shape[1]
    nheads = 12

    h = _sc_gather(emb, x[0])[None] * math.sqrt(d)
    h = h + _pos_encoding(S, d)[None]
    mask = jnp.triu(jnp.full((S, S), -jnp.inf, dtype=jnp.float32), k=1)

    h = _layer_norm(h + _mha(h, l0_in_w, l0_in_b, l0_out_w, l0_out_b, nheads, mask),
                    l0_n1_g, l0_n1_b)
    ff = _bmm(jax.nn.relu(_bmm(h, l0_ff1_w.T) + l0_ff1_b), l0_ff2_w.T) + l0_ff2_b
    h = _layer_norm(h + ff, l0_n2_g, l0_n2_b)

    h = _layer_norm(h + _mha(h, l1_in_w, l1_in_b, l1_out_w, l1_out_b, nheads, mask),
                    l1_n1_g, l1_n1_b)
    xh = h[0]
    ep, gs = _routing_call(xh, l1_gate)
    moe = _moe_call(xh, ep, gs, l1_w1, l1_w2)
    h = _layer_norm(h + moe[None], l1_n2_g, l1_n2_b)

    nll = _decoder_call(h[0], dec_w, dec_b, y[0])
    return nll
